# baseline (speedup 1.0000x reference)
import functools

import jax
import jax.numpy as jnp
from jax import lax
from jax.experimental import pallas as pl
from jax.experimental.pallas import tpu as pltpu
from jax.experimental.pallas import tpu_sc as plsc


_BM = 256


def _gcn_body(np_, adj_ref, feat_ref, w_ref, b_ref, xt_ref, support_ref):
    i = pl.program_id(0)

    @pl.when(i == 0)
    def _():
        support_ref[...] = jnp.dot(
            feat_ref[...], w_ref[...], preferred_element_type=jnp.float32
        )

    acc = jnp.dot(adj_ref[...], support_ref[...], preferred_element_type=jnp.float32)
    xt_blk = jnp.tanh(acc + b_ref[...]).T
    emb = xt_blk.shape[0]
    for d in range(emb):
        xt_ref[pl.ds(d * np_ + i * _BM, _BM)] = xt_blk[d]


def _gcn_xt(feature, adj, W, b, np_):
    n, feat = feature.shape
    emb = W.shape[1]
    grid = (n + _BM - 1) // _BM
    return pl.pallas_call(
        functools.partial(_gcn_body, np_),
        grid=(grid,),
        in_specs=[
            pl.BlockSpec((_BM, n), lambda i: (i, 0)),
            pl.BlockSpec((n, feat), lambda i: (0, 0)),
            pl.BlockSpec((feat, emb), lambda i: (0, 0)),
            pl.BlockSpec((1, emb), lambda i: (0, 0)),
        ],
        out_specs=pl.BlockSpec((emb * np_,), lambda i: (0,)),
        out_shape=jax.ShapeDtypeStruct((emb * np_,), jnp.float32),
        scratch_shapes=[pltpu.VMEM((n, emb), jnp.float32)],
    )(adj, feature, W, b.reshape(1, emb))


_L = 16


def _make_fm(n, emb, batch):
    info = plsc.get_sparse_core_info()
    nc, ns = info.num_cores, info.num_subcores
    d_per_t = emb // ns
    assert d_per_t == 2, "kernel specialised to two dims per subcore"
    bh = batch // nc
    sl = bh // ns
    nvec = bh // _L

    mesh = plsc.VectorSubcoreMesh(core_axis_name="c", subcore_axis_name="s")

    @functools.partial(
        pl.kernel,
        mesh=mesh,
        compiler_params=pltpu.CompilerParams(needs_layout_passes=False),
        out_type=(
            jax.ShapeDtypeStruct((batch,), jnp.float32),
            jax.ShapeDtypeStruct((batch,), jnp.float32),
        ),
        scratch_types=[
            pltpu.VMEM((n,), jnp.float32),
            pltpu.VMEM((n,), jnp.float32),
            pltpu.VMEM((bh,), jnp.int32),
            pltpu.VMEM((bh,), jnp.int32),
            pltpu.VMEM((bh,), jnp.int32),
            pltpu.VMEM((bh,), jnp.float32),
            pltpu.VMEM((bh,), jnp.float32),
            pltpu.VMEM((sl,), jnp.float32),
            pltpu.VMEM((sl,), jnp.float32),
            pltpu.VMEM((ns, sl), jnp.float32),
            pltpu.VMEM((ns, sl), jnp.float32),
            pltpu.VMEM_SHARED((ns, bh), jnp.float32),
            pltpu.VMEM_SHARED((ns, bh), jnp.float32),
            pltpu.SemaphoreType.DMA,
            pltpu.SemaphoreType.DMA,
            pltpu.SemaphoreType.DMA,
            pltpu.SemaphoreType.DMA,
            pltpu.SemaphoreType.DMA,
        ],
    )
    def fm(xt_hbm, u_hbm, p_hbm, nn_hbm, outp_hbm, outn_hbm,
           xloc0, xloc1, u_v, p_v, n_v, part_p, part_n, accp, accn,
           redp, redn, shp, shn, sem0, sem1, sem2, sem3, sem4):
        c = lax.axis_index("c")
        s = lax.axis_index("s")

        cp0 = pltpu.async_copy(
            xt_hbm.at[pl.ds((2 * s) * n, n)], xloc0, sem0)
        cp4 = pltpu.async_copy(
            xt_hbm.at[pl.ds((2 * s + 1) * n, n)], xloc1, sem4)
        cp1 = pltpu.async_copy(u_hbm.at[pl.ds(c * bh, bh)], u_v, sem1)
        cp2 = pltpu.async_copy(p_hbm.at[pl.ds(c * bh, bh)], p_v, sem2)
        cp3 = pltpu.async_copy(nn_hbm.at[pl.ds(c * bh, bh)], n_v, sem3)
        cp0.wait()
        cp4.wait()
        cp1.wait()
        cp2.wait()
        cp3.wait()

        @plsc.parallel_loop(0, nvec, unroll=8)
        def _gather_body(k):
            kv = pl.ds(k * _L, _L)
            u16 = u_v[kv]
            p16 = p_v[kv]
            n16 = n_v[kv]
            ap = jnp.zeros((_L,), jnp.float32)
            an = jnp.zeros((_L,), jnp.float32)
            for xloc in (xloc0, xloc1):
                xu = plsc.load_gather(xloc, [u16])
                xp = plsc.load_gather(xloc, [p16])
                xn = plsc.load_gather(xloc, [n16])
                ap = ap + xu * xp
                an = an + xu * xn
            part_p[kv] = ap
            part_n[kv] = an

        pltpu.sync_copy(part_p, shp.at[s])
        pltpu.sync_copy(part_n, shn.at[s])
        plsc.subcore_barrier()

        base = s * sl
        pltpu.sync_copy(shp.at[:, pl.ds(base, sl)], redp)
        pltpu.sync_copy(shn.at[:, pl.ds(base, sl)], redn)

        @plsc.parallel_loop(0, sl // _L, unroll=2)
        def _reduce_body(v):
            vv = pl.ds(v * _L, _L)
            ap = redp[0, vv]
            an = redn[0, vv]
            for r in range(1, ns):
                ap = ap + redp[r, vv]
                an = an + redn[r, vv]
            accp[vv] = ap
            accn[vv] = an

        out0 = c * bh + base
        pltpu.sync_copy(accp, outp_hbm.at[pl.ds(out0, sl)])
        pltpu.sync_copy(accn, outn_hbm.at[pl.ds(out0, sl)])

    return fm


def kernel(feature, adj, user, item_p, item_n, W, b):
    n, _ = feature.shape
    emb = W.shape[1]
    batch = user.shape[0]
    np_ = ((n + _BM - 1) // _BM) * _BM
    xt = _gcn_xt(feature, adj, W, b, np_)
    fm = _make_fm(np_, emb, batch)
    pred_p, pred_n = fm(xt, user, item_p, item_n)
    return (pred_p, pred_n)


# speedup vs baseline: 1.0058x; 1.0058x over previous
import functools

import jax
import jax.numpy as jnp
from jax import lax
from jax.experimental import pallas as pl
from jax.experimental.pallas import tpu as pltpu
from jax.experimental.pallas import tpu_sc as plsc


_BM = 256


def _gcn_body(np_, adj_ref, feat_ref, w_ref, b_ref, xt_ref, support_ref):
    i = pl.program_id(0)

    @pl.when(i == 0)
    def _():
        support_ref[...] = jnp.dot(
            feat_ref[...], w_ref[...], preferred_element_type=jnp.float32
        )

    acc = jnp.dot(adj_ref[...], support_ref[...], preferred_element_type=jnp.float32)
    xt_blk = jnp.tanh(acc + b_ref[...]).T
    emb = xt_blk.shape[0]
    for d in range(emb):
        xt_ref[pl.ds(d * np_ + i * _BM, _BM)] = xt_blk[d]


def _gcn_xt(feature, adj, W, b, np_):
    n, feat = feature.shape
    emb = W.shape[1]
    grid = (n + _BM - 1) // _BM
    return pl.pallas_call(
        functools.partial(_gcn_body, np_),
        grid=(grid,),
        in_specs=[
            pl.BlockSpec((_BM, n), lambda i: (i, 0)),
            pl.BlockSpec((n, feat), lambda i: (0, 0)),
            pl.BlockSpec((feat, emb), lambda i: (0, 0)),
            pl.BlockSpec((1, emb), lambda i: (0, 0)),
        ],
        out_specs=pl.BlockSpec((emb * np_,), lambda i: (0,)),
        out_shape=jax.ShapeDtypeStruct((emb * np_,), jnp.float32),
        scratch_shapes=[pltpu.VMEM((n, emb), jnp.float32)],
    )(adj, feature, W, b.reshape(1, emb))


_L = 16


def _make_fm(n, emb, batch):
    info = plsc.get_sparse_core_info()
    nc, ns = info.num_cores, info.num_subcores
    d_per_t = emb // ns
    bh = batch // nc
    sl = bh // ns
    nvec = bh // _L

    mesh = plsc.VectorSubcoreMesh(core_axis_name="c", subcore_axis_name="s")

    @functools.partial(
        pl.kernel,
        mesh=mesh,
        compiler_params=pltpu.CompilerParams(needs_layout_passes=False),
        out_type=(
            jax.ShapeDtypeStruct((batch,), jnp.float32),
            jax.ShapeDtypeStruct((batch,), jnp.float32),
        ),
        scratch_types=[
            pltpu.VMEM((d_per_t * n,), jnp.float32),
            pltpu.VMEM((bh,), jnp.int32),
            pltpu.VMEM((bh,), jnp.int32),
            pltpu.VMEM((bh,), jnp.int32),
            pltpu.VMEM((bh,), jnp.float32),
            pltpu.VMEM((bh,), jnp.float32),
            pltpu.VMEM((sl,), jnp.float32),
            pltpu.VMEM((sl,), jnp.float32),
            pltpu.VMEM((ns, sl), jnp.float32),
            pltpu.VMEM((ns, sl), jnp.float32),
            pltpu.VMEM_SHARED((ns, bh), jnp.float32),
            pltpu.VMEM_SHARED((ns, bh), jnp.float32),
            pltpu.SemaphoreType.DMA,
            pltpu.SemaphoreType.DMA,
            pltpu.SemaphoreType.DMA,
            pltpu.SemaphoreType.DMA,
        ],
    )
    def fm(xt_hbm, u_hbm, p_hbm, nn_hbm, outp_hbm, outn_hbm,
           xloc, u_v, p_v, n_v, part_p, part_n, accp, accn, redp, redn,
           shp, shn, sem0, sem1, sem2, sem3):
        c = lax.axis_index("c")
        s = lax.axis_index("s")

        cp0 = pltpu.async_copy(
            xt_hbm.at[pl.ds(s * (d_per_t * n), d_per_t * n)], xloc, sem0)
        cp1 = pltpu.async_copy(u_hbm.at[pl.ds(c * bh, bh)], u_v, sem1)
        cp2 = pltpu.async_copy(p_hbm.at[pl.ds(c * bh, bh)], p_v, sem2)
        cp3 = pltpu.async_copy(nn_hbm.at[pl.ds(c * bh, bh)], n_v, sem3)
        cp0.wait()
        cp1.wait()
        cp2.wait()
        cp3.wait()

        @plsc.parallel_loop(0, nvec, unroll=8)
        def _gather_body(k):
            kv = pl.ds(k * _L, _L)
            u16 = u_v[kv]
            p16 = p_v[kv]
            n16 = n_v[kv]
            ap = jnp.zeros((_L,), jnp.float32)
            an = jnp.zeros((_L,), jnp.float32)
            for dl in range(d_per_t):
                off = dl * n
                xu = plsc.load_gather(xloc, [u16 + off])
                xp = plsc.load_gather(xloc, [p16 + off])
                xn = plsc.load_gather(xloc, [n16 + off])
                ap = ap + xu * xp
                an = an + xu * xn
            part_p[kv] = ap
            part_n[kv] = an

        pltpu.sync_copy(part_p, shp.at[s])
        pltpu.sync_copy(part_n, shn.at[s])
        plsc.subcore_barrier()

        base = s * sl
        pltpu.sync_copy(shp.at[:, pl.ds(base, sl)], redp)
        pltpu.sync_copy(shn.at[:, pl.ds(base, sl)], redn)

        @plsc.parallel_loop(0, sl // _L, unroll=2)
        def _reduce_body(v):
            vv = pl.ds(v * _L, _L)
            ap = redp[0, vv]
            an = redn[0, vv]
            for r in range(1, ns):
                ap = ap + redp[r, vv]
                an = an + redn[r, vv]
            accp[vv] = ap
            accn[vv] = an

        out0 = c * bh + base
        pltpu.sync_copy(accp, outp_hbm.at[pl.ds(out0, sl)])
        pltpu.sync_copy(accn, outn_hbm.at[pl.ds(out0, sl)])

    return fm


def kernel(feature, adj, user, item_p, item_n, W, b):
    n, _ = feature.shape
    emb = W.shape[1]
    batch = user.shape[0]
    np_ = ((n + _BM - 1) // _BM) * _BM
    xt = _gcn_xt(feature, adj, W, b, np_)
    fm = _make_fm(np_, emb, batch)
    pred_p, pred_n = fm(xt, user, item_p, item_n)
    return (pred_p, pred_n)
